# baseline (speedup 1.0000x reference)
import functools

import jax
import jax.numpy as jnp
from jax import lax
from jax.experimental import pallas as pl
from jax.experimental.pallas import tpu as pltpu
from jax.experimental.pallas import tpu_sc as plsc

NUM_CLUSTERS = 1000000
Z_N = 16
B, L = 4096, 200

NC, NS = 2, 16
NW = NC * NS
TOTAL = B * L
PER_W = TOTAL // NW
IW = 128
NROW = PER_W // IW
G = 4
GSZ = G * IW
NG = NROW // G


def _gather_body(x_hbm, mu_hbm, lv_hbm, z_out, mu_out, lv_out,
                 idx_v, mu_buf, lv_buf, sem):
    wid = lax.axis_index("s") * NC + lax.axis_index("c")
    base = wid * PER_W
    pltpu.sync_copy(x_hbm.at[wid], idx_v)

    def group(g, carry):
        descs = []
        for j in range(G):
            r = g * G + j
            dst = pl.ds(j * IW, IW)
            descs.append(
                pltpu.async_copy(mu_hbm.at[idx_v.at[r]], mu_buf.at[dst], sem))
            descs.append(
                pltpu.async_copy(lv_hbm.at[idx_v.at[r]], lv_buf.at[dst], sem))
        for d in descs:
            d.wait()
        out_sl = pl.ds(base + g * GSZ, GSZ)
        pltpu.sync_copy(mu_buf, z_out.at[out_sl])
        pltpu.sync_copy(mu_buf, mu_out.at[out_sl])
        pltpu.sync_copy(lv_buf, lv_out.at[out_sl])
        return carry

    lax.fori_loop(0, NG, group, 0)


@jax.jit
def _run(x3, embed_mu, embed_logvar):
    mesh = plsc.VectorSubcoreMesh(core_axis_name="c", subcore_axis_name="s")
    kfn = pl.kernel(
        _gather_body,
        out_type=(
            jax.ShapeDtypeStruct((TOTAL, Z_N), jnp.float32),
            jax.ShapeDtypeStruct((TOTAL, Z_N), jnp.float32),
            jax.ShapeDtypeStruct((TOTAL, Z_N), jnp.float32),
        ),
        mesh=mesh,
        compiler_params=pltpu.CompilerParams(use_tc_tiling_on_sc=False),
        scratch_types=[
            pltpu.VMEM((NROW, IW), jnp.int32),
            pltpu.VMEM((GSZ, Z_N), jnp.float32),
            pltpu.VMEM((GSZ, Z_N), jnp.float32),
            pltpu.SemaphoreType.DMA,
        ],
    )
    return kfn(x3, embed_mu, embed_logvar)


def kernel(x, embed_mu, embed_logvar):
    x3 = x.reshape(NW, NROW, IW).astype(jnp.int32)
    z_r, mu_r, lv_r = _run(x3, embed_mu, embed_logvar)
    z = z_r.reshape(B, L, Z_N)
    mu = mu_r.reshape(B, L, Z_N)
    logvar = lv_r.reshape(B, L, Z_N)
    return (z, mu, logvar)
